# baseline (speedup 1.0000x reference)
import functools

import jax
import jax.numpy as jnp
from jax import lax
from jax.experimental import pallas as pl
from jax.experimental.pallas import tpu as pltpu
from jax.experimental.pallas import tpu_sc as plsc

NUM_CORES = 2
NUM_SUBCORES = 16
NUM_WORKERS = NUM_CORES * NUM_SUBCORES
CHUNK = 128


@functools.partial(jax.jit, static_argnames=("batch", "dim"))
def _lookup_one(idx, weight, *, batch, dim):
    b_per_w = batch // NUM_WORKERS
    n_chunks = b_per_w // CHUNK

    mesh = plsc.VectorSubcoreMesh(core_axis_name="c", subcore_axis_name="s")

    @functools.partial(
        pl.kernel,
        mesh=mesh,
        out_type=jax.ShapeDtypeStruct((NUM_WORKERS, batch // NUM_WORKERS, dim),
                                      jnp.float32),
        scratch_types=[
            pltpu.VMEM((b_per_w,), jnp.int32),
            pltpu.VMEM((b_per_w, dim), jnp.float32),
            pltpu.SemaphoreType.DMA,
        ],
        compiler_params=pltpu.CompilerParams(use_tc_tiling_on_sc=False),
    )
    def gather_kernel(idx_hbm, w_hbm, out_hbm, idx_v, rows_v, sem):
        wid = lax.axis_index("s") * NUM_CORES + lax.axis_index("c")
        base = wid * b_per_w
        pltpu.sync_copy(idx_hbm.at[pl.ds(base, b_per_w)], idx_v)
        copies = []
        for j in range(n_chunks):
            sl = pl.ds(j * CHUNK, CHUNK)
            copies.append(
                pltpu.async_copy(w_hbm.at[idx_v.at[sl]], rows_v.at[sl], sem))
        for c in copies:
            c.wait()
        pltpu.sync_copy(rows_v, out_hbm.at[wid])

    return gather_kernel(idx, weight)


def kernel(user, item, user_weight, item_weight):
    batch = user.shape[0]
    dim = user_weight.shape[1]
    u_flat = _lookup_one(user.astype(jnp.int32), user_weight,
                         batch=batch, dim=dim)
    i_flat = _lookup_one(item.astype(jnp.int32), item_weight,
                         batch=batch, dim=dim)
    return (u_flat.reshape(batch, dim), i_flat.reshape(batch, dim))


# speedup vs baseline: 1.5651x; 1.5651x over previous
import functools

import jax
import jax.numpy as jnp
from jax import lax
from jax.experimental import pallas as pl
from jax.experimental.pallas import tpu as pltpu
from jax.experimental.pallas import tpu_sc as plsc

NUM_CORES = 2
NUM_SUBCORES = 16
NUM_WORKERS = NUM_CORES * NUM_SUBCORES


@functools.partial(jax.jit, static_argnames=("batch", "dim"))
def _lookup(user, item, user_weight, item_weight, *, batch, dim):
    b_per_w = batch // NUM_WORKERS
    pairs_per_w = b_per_w // 2

    mesh = plsc.VectorSubcoreMesh(core_axis_name="c", subcore_axis_name="s")

    @functools.partial(
        pl.kernel,
        mesh=mesh,
        out_type=(
            jax.ShapeDtypeStruct((batch // 2, 2 * dim), jnp.float32),
            jax.ShapeDtypeStruct((batch // 2, 2 * dim), jnp.float32),
        ),
        scratch_types=[
            pltpu.VMEM((b_per_w,), jnp.int32),
            pltpu.VMEM((b_per_w,), jnp.int32),
            pltpu.VMEM((pairs_per_w, 2 * dim), jnp.float32),
            pltpu.VMEM((pairs_per_w, 2 * dim), jnp.float32),
            pltpu.SemaphoreType.DMA,
        ],
    )
    def gather_kernel(user_hbm, item_hbm, uw_hbm, iw_hbm,
                      out_u_hbm, out_i_hbm,
                      idx_u, idx_i, rows_u, rows_i, gsem):
        wid = lax.axis_index("s") * NUM_CORES + lax.axis_index("c")
        base = wid * b_per_w
        pltpu.sync_copy(user_hbm.at[pl.ds(base, b_per_w)], idx_u)
        pltpu.sync_copy(item_hbm.at[pl.ds(base, b_per_w)], idx_i)

        def fire(g, carry):
            b = g * 16
            vu = idx_u[pl.ds(b, 16)]
            vi = idx_i[pl.ds(b, 16)]
            for k in range(16):
                j2 = g * 8 + (k // 2)
                off = (k % 2) * dim
                pltpu.make_async_copy(
                    uw_hbm.at[vu[k]],
                    rows_u.at[j2, pl.ds(off, dim)], gsem).start()
                pltpu.make_async_copy(
                    iw_hbm.at[vi[k]],
                    rows_i.at[j2, pl.ds(off, dim)], gsem).start()
            return carry

        lax.fori_loop(0, b_per_w // 16, fire, 0)
        pltpu.make_async_copy(
            out_u_hbm.at[pl.ds(0, pairs_per_w)], rows_u, gsem).wait()
        pltpu.make_async_copy(
            out_i_hbm.at[pl.ds(0, pairs_per_w)], rows_i, gsem).wait()
        pltpu.sync_copy(rows_u, out_u_hbm.at[pl.ds(wid * pairs_per_w,
                                                   pairs_per_w)])
        pltpu.sync_copy(rows_i, out_i_hbm.at[pl.ds(wid * pairs_per_w,
                                                   pairs_per_w)])

    return gather_kernel(user, item, user_weight, item_weight)


def kernel(user, item, user_weight, item_weight):
    batch = user.shape[0]
    dim = user_weight.shape[1]
    user = user.astype(jnp.int32)
    item = item.astype(jnp.int32)
    u2, i2 = _lookup(user, item, user_weight, item_weight,
                     batch=batch, dim=dim)
    return (u2.reshape(batch, dim), i2.reshape(batch, dim))
